# baseline (speedup 1.0000x reference)
import jax
import jax.numpy as jnp
from jax import lax
from jax.experimental import pallas as pl
from jax.experimental.pallas import tpu as pltpu
from jax.experimental.pallas import tpu_sc as plsc

N = 10000
E = 320000
D = 128
K = 3

NC = 2
NS = 16
L = 16
DH = D // NC
NP = 10240
RPT = NP // NS
NCH = 16
CH = RPT // NCH
EB = 256
NB = -(-E // (NS * EB))
W = 4
NR = 4
NWIN = -(-NB // W)
NBP = NWIN * W
NG = EB // L


def _fold_body(xs_hbm, alpha_hbm, rows_hbm, cols_hbm, w_hbm,
               y_out, za, zb,
               cw2, rw2, ww2, rbuf, stage2, ybuf2, zero_v,
               alpha_v,
               gsem, ssem, isem, sa, sy, sb, sf, sc,
               acc_sh):
  c = lax.axis_index("c")
  s = lax.axis_index("s")
  row0 = s * RPT

  pltpu.sync_copy(alpha_hbm, alpha_v)
  av = alpha_v[...]
  m = av[0]
  for i in range(1, K + 1):
    m = jnp.maximum(m, av[i])
  ev = jnp.exp(av - m)
  ssum = ev[0]
  for i in range(1, K + 1):
    ssum = ssum + ev[i]
  anorm = ev / ssum

  def _zinit(r, _):
    for j in range(DH // L):
      zero_v[r, pl.ds(j * L, L)] = jnp.zeros((L,), jnp.float32)
    return 0
  lax.fori_loop(0, CH, _zinit, 0)

  a0 = anorm[0]
  for t in range(NCH):
    rsl = pl.ds(row0 + t * CH, CH)
    pltpu.sync_copy(zero_v, acc_sh.at[rsl])
    stage_v = stage2.at[pl.ds((t % 2) * CH, CH)]
    ybuf = ybuf2.at[pl.ds((t % 2) * CH, CH)]
    pltpu.sync_copy(xs_hbm.at[c, rsl], stage_v)
    def _scale0(r, _):
      for j in range(DH // L):
        sl = pl.ds(j * L, L)
        ybuf[r, sl] = stage_v[r, sl] * a0
      return 0
    lax.fori_loop(0, CH, _scale0, 0)
    pltpu.sync_copy(ybuf, y_out.at[c, rsl])

  plsc.subcore_barrier()

  srcs = (xs_hbm, za, zb)
  dsts = (za, zb, None)
  for hop in range(K):
    src = srcs[hop].at[c]
    ak = anorm[hop + 1]

    pltpu.async_copy(cols_hbm.at[s, pl.ds(0, W)], cw2.at[pl.ds(0, W)], isem)
    pltpu.async_copy(rows_hbm.at[s, pl.ds(0, W)], rw2.at[pl.ds(0, W)], isem)
    pltpu.async_copy(w_hbm.at[s, pl.ds(0, W * NG)], ww2.at[pl.ds(0, W * NG)],
                     isem)

    def _window(wn, _):
      p = lax.rem(wn, 2)
      pb = p * W
      qb = (1 - p) * W
      pltpu.make_async_copy(
          cols_hbm.at[s, pl.ds(0, W)], cw2.at[pl.ds(0, W)], isem).wait()
      pltpu.make_async_copy(
          rows_hbm.at[s, pl.ds(0, W)], rw2.at[pl.ds(0, W)], isem).wait()
      pltpu.make_async_copy(
          w_hbm.at[s, pl.ds(0, W * NG)], ww2.at[pl.ds(0, W * NG)], isem).wait()

      @pl.when(wn > 0)
      def _():
        for i in range(W // 2):
          pltpu.make_async_copy(
              rbuf.at[pl.ds(i * EB, EB)], acc_sh.at[rw2.at[pb + i]],
              ssem.at[i]).wait()
      for i in range(W // 2):
        pltpu.async_copy(src.at[cw2.at[pb + i]],
                         rbuf.at[pl.ds(i * EB, EB)], gsem.at[i])
      @pl.when(wn > 0)
      def _():
        for i in range(W // 2, W):
          pltpu.make_async_copy(
              rbuf.at[pl.ds(i * EB, EB)], acc_sh.at[rw2.at[pb + i]],
              ssem.at[i]).wait()
      for i in range(W // 2, W):
        pltpu.async_copy(src.at[cw2.at[pb + i]],
                         rbuf.at[pl.ds(i * EB, EB)], gsem.at[i])

      b1 = (wn + 1) * W
      pltpu.async_copy(cols_hbm.at[s, pl.ds(b1, W)], cw2.at[pl.ds(qb, W)],
                       isem)
      pltpu.async_copy(rows_hbm.at[s, pl.ds(b1, W)], rw2.at[pl.ds(qb, W)],
                       isem)
      pltpu.async_copy(w_hbm.at[s, pl.ds(b1 * NG, W * NG)],
                       ww2.at[pl.ds(qb * NG, W * NG)], isem)

      for i in range(W):
        pltpu.make_async_copy(src.at[cw2.at[pb + i]],
                              rbuf.at[pl.ds(i * EB, EB)], gsem.at[i]).wait()
        def _grp(g, _):
          wv = ww2[(pb + i) * NG + g]
          e0 = i * EB + g * L
          for le in range(L):
            wsc = wv[le]
            for j in range(DH // L):
              sl = pl.ds(j * L, L)
              rbuf[e0 + le, sl] = rbuf[e0 + le, sl] * wsc
          return 0
        lax.fori_loop(0, NG, _grp, 0)
        pltpu.async_copy(rbuf.at[pl.ds(i * EB, EB)],
                         acc_sh.at[rw2.at[pb + i]], ssem.at[i], add=True)
      return 0
    lax.fori_loop(0, NWIN, _window, 0)

    for i in range(W):
      pltpu.make_async_copy(rbuf.at[pl.ds(i * EB, EB)],
                            acc_sh.at[rw2.at[i]], ssem.at[i]).wait()
    pltpu.make_async_copy(
        cols_hbm.at[s, pl.ds(0, W)], cw2.at[pl.ds(0, W)], isem).wait()
    pltpu.make_async_copy(
        rows_hbm.at[s, pl.ds(0, W)], rw2.at[pl.ds(0, W)], isem).wait()
    pltpu.make_async_copy(
        w_hbm.at[s, pl.ds(0, W * NG)], ww2.at[pl.ds(0, W * NG)], isem).wait()

    plsc.subcore_barrier()

    dst = dsts[hop]

    def _chunk_refs(t):
      p = t % 2
      rsl = pl.ds(row0 + t * CH, CH)
      stage_v = stage2.at[pl.ds(p * CH, CH)]
      ybuf = ybuf2.at[pl.ds(p * CH, CH)]
      return p, rsl, stage_v, ybuf

    def _issue_loads(t):
      p, rsl, stage_v, ybuf = _chunk_refs(t)
      pltpu.async_copy(acc_sh.at[rsl], stage_v, sa.at[p])
      pltpu.async_copy(y_out.at[c, rsl], ybuf, sy.at[p])

    _issue_loads(0)
    for t in range(NCH):
      p, rsl, stage_v, ybuf = _chunk_refs(t)
      if t >= 1:
        p1, rsl1, stage_v1, ybuf1 = _chunk_refs(t - 1)
        if dst is not None:
          pltpu.make_async_copy(stage_v1, dst.at[c, rsl1], sb.at[p1]).wait()
        pltpu.make_async_copy(ybuf1, y_out.at[c, rsl1], sf.at[p1]).wait()
      if t + 1 < NCH:
        _issue_loads(t + 1)
      pltpu.make_async_copy(acc_sh.at[rsl], stage_v, sa.at[p]).wait()
      if dst is not None:
        pltpu.async_copy(stage_v, dst.at[c, rsl], sb.at[p])
        pltpu.async_copy(zero_v, acc_sh.at[rsl], sc)
      pltpu.make_async_copy(y_out.at[c, rsl], ybuf, sy.at[p]).wait()
      def _scalek(r, _):
        for j in range(DH // L):
          sl = pl.ds(j * L, L)
          ybuf[r, sl] = ybuf[r, sl] + stage_v[r, sl] * ak
        return 0
      lax.fori_loop(0, CH, _scalek, 0)
      pltpu.async_copy(ybuf, y_out.at[c, rsl], sf.at[p])
    p, rsl, stage_v, ybuf = _chunk_refs(NCH - 1)
    if dst is not None:
      pltpu.make_async_copy(stage_v, dst.at[c, rsl], sb.at[p]).wait()
    pltpu.make_async_copy(ybuf, y_out.at[c, rsl], sf.at[p]).wait()
    if dst is not None:
      for t in range(NCH):
        _, rsl, _, _ = _chunk_refs(t)
        pltpu.make_async_copy(zero_v, acc_sh.at[rsl], sc).wait()
      plsc.subcore_barrier()


_fold = pl.kernel(
    _fold_body,
    out_type=(
        jax.ShapeDtypeStruct((NC, NP, DH), jnp.float32),
        jax.ShapeDtypeStruct((NC, NP, DH), jnp.float32),
        jax.ShapeDtypeStruct((NC, NP, DH), jnp.float32),
    ),
    mesh=plsc.VectorSubcoreMesh(core_axis_name="c", subcore_axis_name="s"),
    compiler_params=pltpu.CompilerParams(use_tc_tiling_on_sc=False),
    scratch_types=[
        pltpu.VMEM((2 * W, EB), jnp.int32),
        pltpu.VMEM((2 * W, EB), jnp.int32),
        pltpu.VMEM((2 * W * NG, L), jnp.float32),
        pltpu.VMEM((W * EB, DH), jnp.float32),
        pltpu.VMEM((2 * CH, DH), jnp.float32),
        pltpu.VMEM((2 * CH, DH), jnp.float32),
        pltpu.VMEM((CH, DH), jnp.float32),
        pltpu.VMEM((L,), jnp.float32),
        pltpu.SemaphoreType.DMA((W,)),
        pltpu.SemaphoreType.DMA((W,)),
        pltpu.SemaphoreType.DMA,
        pltpu.SemaphoreType.DMA((2,)),
        pltpu.SemaphoreType.DMA((2,)),
        pltpu.SemaphoreType.DMA((2,)),
        pltpu.SemaphoreType.DMA((2,)),
        pltpu.SemaphoreType.DMA,
        pltpu.VMEM_SHARED((NP, DH), jnp.float32),
    ],
)


@jax.jit
def kernel(x, edge_index, edge_weight, alpha_logits):
  xp = jnp.pad(x, ((0, NP - N), (0, 0)))
  xs = jnp.stack([xp[:, :DH], xp[:, DH:]])
  row = edge_index[0].astype(jnp.int32)
  col = edge_index[1].astype(jnp.int32)
  w = edge_weight.astype(jnp.float32)
  pad = NS * NBP * EB - E
  row = jnp.concatenate([row, jnp.zeros((pad,), jnp.int32)]).reshape(NS, NBP, EB)
  col = jnp.concatenate([col, jnp.zeros((pad,), jnp.int32)]).reshape(NS, NBP, EB)
  w = jnp.concatenate([w, jnp.zeros((pad,), jnp.float32)]).reshape(NS, NBP * NG, L)
  row = jnp.pad(row, ((0, 0), (0, W), (0, 0)))
  col = jnp.pad(col, ((0, 0), (0, W), (0, 0)))
  w = jnp.pad(w, ((0, 0), (0, W * NG), (0, 0)))
  alpha_pad = jnp.concatenate(
      [alpha_logits.astype(jnp.float32), jnp.full((L - K - 1,), -1e30, jnp.float32)])
  y, _, _ = _fold(xs, alpha_pad, row, col, w)
  return jnp.concatenate([y[0, :N], y[1, :N]], axis=1)


# speedup vs baseline: 1.5278x; 1.5278x over previous
import jax
import jax.numpy as jnp
from jax import lax
from jax.experimental import pallas as pl
from jax.experimental.pallas import tpu as pltpu
from jax.experimental.pallas import tpu_sc as plsc

N = 10000
E = 320000
D = 128
K = 3

NC = 2
NS = 16
L = 16
DH = D // NC
NP = 10240
RPT = NP // NS
NCH = 32
CH = RPT // NCH
EB = 128
NB = -(-E // (NS * EB))
W = 4
NR = 4
NWIN = -(-NB // W)
NBP = NWIN * W
NG = EB // L


def _fold_body(xs_hbm, alpha_hbm, rows_hbm, cols_hbm, w_hbm,
               y_out,
               cw2, rw2, ww2, rbuf, stage2, ybuf2, zero_v,
               alpha_v,
               gsem, ssem, isem, sa, sy, sb, sf, sc,
               acc_sh, src_sh):
  c = lax.axis_index("c")
  s = lax.axis_index("s")
  row0 = s * RPT

  pltpu.sync_copy(alpha_hbm, alpha_v)
  av = alpha_v[...]
  m = av[0]
  for i in range(1, K + 1):
    m = jnp.maximum(m, av[i])
  ev = jnp.exp(av - m)
  ssum = ev[0]
  for i in range(1, K + 1):
    ssum = ssum + ev[i]
  anorm = ev / ssum

  def _zinit(r, _):
    for j in range(DH // L):
      zero_v[r, pl.ds(j * L, L)] = jnp.zeros((L,), jnp.float32)
    return 0
  lax.fori_loop(0, CH, _zinit, 0)

  a0 = anorm[0]
  for t in range(NCH):
    rsl = pl.ds(row0 + t * CH, CH)
    pltpu.sync_copy(zero_v, acc_sh.at[rsl])
    stage_v = stage2.at[pl.ds((t % 2) * CH, CH)]
    ybuf = ybuf2.at[pl.ds((t % 2) * CH, CH)]
    pltpu.sync_copy(xs_hbm.at[c, rsl], stage_v)
    pltpu.sync_copy(stage_v, src_sh.at[rsl])
    def _scale0(r, _):
      for j in range(DH // L):
        sl = pl.ds(j * L, L)
        ybuf[r, sl] = stage_v[r, sl] * a0
      return 0
    lax.fori_loop(0, CH, _scale0, 0)
    pltpu.sync_copy(ybuf, y_out.at[c, rsl])

  plsc.subcore_barrier()

  for hop in range(K):
    src = src_sh
    dst = src_sh if hop + 1 < K else None
    ak = anorm[hop + 1]

    pltpu.async_copy(cols_hbm.at[s, pl.ds(0, W)], cw2.at[pl.ds(0, W)], isem)
    pltpu.async_copy(rows_hbm.at[s, pl.ds(0, W)], rw2.at[pl.ds(0, W)], isem)
    pltpu.async_copy(w_hbm.at[s, pl.ds(0, W * NG)], ww2.at[pl.ds(0, W * NG)],
                     isem)

    def _window(wn, _):
      p = lax.rem(wn, 2)
      pb = p * W
      qb = (1 - p) * W
      pltpu.make_async_copy(
          cols_hbm.at[s, pl.ds(0, W)], cw2.at[pl.ds(0, W)], isem).wait()
      pltpu.make_async_copy(
          rows_hbm.at[s, pl.ds(0, W)], rw2.at[pl.ds(0, W)], isem).wait()
      pltpu.make_async_copy(
          w_hbm.at[s, pl.ds(0, W * NG)], ww2.at[pl.ds(0, W * NG)], isem).wait()

      @pl.when(wn > 0)
      def _():
        for i in range(W // 2):
          pltpu.make_async_copy(
              rbuf.at[pl.ds(i * EB, EB)], acc_sh.at[rw2.at[pb + i]],
              ssem.at[i]).wait()
      for i in range(W // 2):
        pltpu.async_copy(src.at[cw2.at[pb + i]],
                         rbuf.at[pl.ds(i * EB, EB)], gsem.at[i])
      @pl.when(wn > 0)
      def _():
        for i in range(W // 2, W):
          pltpu.make_async_copy(
              rbuf.at[pl.ds(i * EB, EB)], acc_sh.at[rw2.at[pb + i]],
              ssem.at[i]).wait()
      for i in range(W // 2, W):
        pltpu.async_copy(src.at[cw2.at[pb + i]],
                         rbuf.at[pl.ds(i * EB, EB)], gsem.at[i])

      b1 = (wn + 1) * W
      pltpu.async_copy(cols_hbm.at[s, pl.ds(b1, W)], cw2.at[pl.ds(qb, W)],
                       isem)
      pltpu.async_copy(rows_hbm.at[s, pl.ds(b1, W)], rw2.at[pl.ds(qb, W)],
                       isem)
      pltpu.async_copy(w_hbm.at[s, pl.ds(b1 * NG, W * NG)],
                       ww2.at[pl.ds(qb * NG, W * NG)], isem)

      for i in range(W):
        pltpu.make_async_copy(src.at[cw2.at[pb + i]],
                              rbuf.at[pl.ds(i * EB, EB)], gsem.at[i]).wait()
        def _grp(g, _):
          wv = ww2[(pb + i) * NG + g]
          e0 = i * EB + g * L
          for le in range(L):
            wsc = wv[le]
            for j in range(DH // L):
              sl = pl.ds(j * L, L)
              rbuf[e0 + le, sl] = rbuf[e0 + le, sl] * wsc
          return 0
        lax.fori_loop(0, NG, _grp, 0)
        pltpu.async_copy(rbuf.at[pl.ds(i * EB, EB)],
                         acc_sh.at[rw2.at[pb + i]], ssem.at[i], add=True)
      return 0
    lax.fori_loop(0, NWIN, _window, 0)

    for i in range(W):
      pltpu.make_async_copy(rbuf.at[pl.ds(i * EB, EB)],
                            acc_sh.at[rw2.at[i]], ssem.at[i]).wait()
    pltpu.make_async_copy(
        cols_hbm.at[s, pl.ds(0, W)], cw2.at[pl.ds(0, W)], isem).wait()
    pltpu.make_async_copy(
        rows_hbm.at[s, pl.ds(0, W)], rw2.at[pl.ds(0, W)], isem).wait()
    pltpu.make_async_copy(
        w_hbm.at[s, pl.ds(0, W * NG)], ww2.at[pl.ds(0, W * NG)], isem).wait()

    plsc.subcore_barrier()

    def _chunk_refs(t):
      p = t % 2
      rsl = pl.ds(row0 + t * CH, CH)
      stage_v = stage2.at[pl.ds(p * CH, CH)]
      ybuf = ybuf2.at[pl.ds(p * CH, CH)]
      return p, rsl, stage_v, ybuf

    def _issue_loads(t):
      p, rsl, stage_v, ybuf = _chunk_refs(t)
      pltpu.async_copy(acc_sh.at[rsl], stage_v, sa.at[p])
      pltpu.async_copy(y_out.at[c, rsl], ybuf, sy.at[p])

    _issue_loads(0)
    for t in range(NCH):
      p, rsl, stage_v, ybuf = _chunk_refs(t)
      if t >= 1:
        p1, rsl1, stage_v1, ybuf1 = _chunk_refs(t - 1)
        if dst is not None:
          pltpu.make_async_copy(stage_v1, dst.at[rsl1], sb.at[p1]).wait()
        pltpu.make_async_copy(ybuf1, y_out.at[c, rsl1], sf.at[p1]).wait()
      if t + 1 < NCH:
        _issue_loads(t + 1)
      pltpu.make_async_copy(acc_sh.at[rsl], stage_v, sa.at[p]).wait()
      if dst is not None:
        pltpu.async_copy(stage_v, dst.at[rsl], sb.at[p])
        pltpu.async_copy(zero_v, acc_sh.at[rsl], sc)
      pltpu.make_async_copy(y_out.at[c, rsl], ybuf, sy.at[p]).wait()
      def _scalek(r, _):
        for j in range(DH // L):
          sl = pl.ds(j * L, L)
          ybuf[r, sl] = ybuf[r, sl] + stage_v[r, sl] * ak
        return 0
      lax.fori_loop(0, CH, _scalek, 0)
      pltpu.async_copy(ybuf, y_out.at[c, rsl], sf.at[p])
    p, rsl, stage_v, ybuf = _chunk_refs(NCH - 1)
    if dst is not None:
      pltpu.make_async_copy(stage_v, dst.at[rsl], sb.at[p]).wait()
    pltpu.make_async_copy(ybuf, y_out.at[c, rsl], sf.at[p]).wait()
    if dst is not None:
      for t in range(NCH):
        _, rsl, _, _ = _chunk_refs(t)
        pltpu.make_async_copy(zero_v, acc_sh.at[rsl], sc).wait()
      plsc.subcore_barrier()


_fold = pl.kernel(
    _fold_body,
    out_type=jax.ShapeDtypeStruct((NC, NP, DH), jnp.float32),
    mesh=plsc.VectorSubcoreMesh(core_axis_name="c", subcore_axis_name="s"),
    compiler_params=pltpu.CompilerParams(use_tc_tiling_on_sc=False),
    scratch_types=[
        pltpu.VMEM((2 * W, EB), jnp.int32),
        pltpu.VMEM((2 * W, EB), jnp.int32),
        pltpu.VMEM((2 * W * NG, L), jnp.float32),
        pltpu.VMEM((W * EB, DH), jnp.float32),
        pltpu.VMEM((2 * CH, DH), jnp.float32),
        pltpu.VMEM((2 * CH, DH), jnp.float32),
        pltpu.VMEM((CH, DH), jnp.float32),
        pltpu.VMEM((L,), jnp.float32),
        pltpu.SemaphoreType.DMA((W,)),
        pltpu.SemaphoreType.DMA((W,)),
        pltpu.SemaphoreType.DMA,
        pltpu.SemaphoreType.DMA((2,)),
        pltpu.SemaphoreType.DMA((2,)),
        pltpu.SemaphoreType.DMA((2,)),
        pltpu.SemaphoreType.DMA((2,)),
        pltpu.SemaphoreType.DMA,
        pltpu.VMEM_SHARED((NP, DH), jnp.float32),
        pltpu.VMEM_SHARED((NP, DH), jnp.float32),
    ],
)


@jax.jit
def kernel(x, edge_index, edge_weight, alpha_logits):
  xp = jnp.pad(x, ((0, NP - N), (0, 0)))
  xs = jnp.stack([xp[:, :DH], xp[:, DH:]])
  row = edge_index[0].astype(jnp.int32)
  col = edge_index[1].astype(jnp.int32)
  w = edge_weight.astype(jnp.float32)
  pad = NS * NBP * EB - E
  row = jnp.concatenate([row, jnp.zeros((pad,), jnp.int32)]).reshape(NS, NBP, EB)
  col = jnp.concatenate([col, jnp.zeros((pad,), jnp.int32)]).reshape(NS, NBP, EB)
  w = jnp.concatenate([w, jnp.zeros((pad,), jnp.float32)]).reshape(NS, NBP * NG, L)
  row = jnp.pad(row, ((0, 0), (0, W), (0, 0)))
  col = jnp.pad(col, ((0, 0), (0, W), (0, 0)))
  w = jnp.pad(w, ((0, 0), (0, W * NG), (0, 0)))
  alpha_pad = jnp.concatenate(
      [alpha_logits.astype(jnp.float32), jnp.full((L - K - 1,), -1e30, jnp.float32)])
  y = _fold(xs, alpha_pad, row, col, w)
  return jnp.concatenate([y[0, :N], y[1, :N]], axis=1)
